# Initial kernel scaffold; baseline (speedup 1.0000x reference)
import jax
import jax.numpy as jnp
from jax.experimental import pallas as pl


def kernel(x_idx, embedding, w_ih_t, w_hh_t, b_rnn, w_fc_t, b_fc):
    raise NotImplementedError("write your pallas kernel here")


# speedup vs baseline: 1.4209x; 1.4209x over previous
import jax
import jax.numpy as jnp
from jax.experimental import pallas as pl
from jax.experimental.pallas import tpu as pltpu


def _round_up(x, m):
    return (x + m - 1) // m * m


def _pad_to(a, shape):
    pads = [(0, s - d) for d, s in zip(a.shape, shape)]
    if all(p == (0, 0) for p in pads):
        return a
    return jnp.pad(a, pads)


def _make_body(total_t, chunk, tb, hp, needs_mask):
    proj_g = 1
    for g in (2, 4):
        if chunk % g == 0:
            proj_g = g

    def body(emb_ref, wih_ref, brnn_ref, whh_ref, wfc_ref, bfc_ref,
             out_ref, h_ref, xp_ref):
        c = pl.program_id(1)

        @pl.when(c == 0)
        def _():
            h_ref[...] = jnp.zeros_like(h_ref)

        wih = wih_ref[...]
        brnn = brnn_ref[...]
        for g in range(chunk // proj_g):
            emb_g = emb_ref[pl.ds(g * proj_g, proj_g)].reshape(proj_g * tb, hp)
            xp_g = jnp.dot(emb_g, wih, preferred_element_type=jnp.float32)
            xp_ref[pl.ds(g * proj_g, proj_g)] = (
                (xp_g + brnn).reshape(proj_g, tb, hp))

        whh = whh_ref[...]
        base = c * chunk
        h = h_ref[...]
        for t in range(chunk):
            pre = xp_ref[t] + jnp.dot(h, whh,
                                      preferred_element_type=jnp.float32)
            h_new = jnp.tanh(pre)
            if needs_mask:
                h_new = jnp.where(base + t < total_t, h_new, h)
            h = h_new
        h_ref[...] = h

        @pl.when(c == pl.num_programs(1) - 1)
        def _():
            out_ref[...] = (
                jnp.dot(h, wfc_ref[...], preferred_element_type=jnp.float32)
                + bfc_ref[...]
            ).astype(out_ref.dtype)

    return body


def kernel(x_idx, embedding, w_ih_t, w_hh_t, b_rnn, w_fc_t, b_fc):
    B, T = x_idx.shape
    H = embedding.shape[1]
    O = w_fc_t.shape[1]

    Hp = _round_up(H, 128)
    Op = _round_up(O, 128)
    Bp = _round_up(B, 8)

    if Bp >= 16 and Bp % 16 == 0:
        n_btiles, tb = 2, Bp // 2
    else:
        n_btiles, tb = 1, Bp

    chunk = 0
    for cand in range(min(T, 32), 0, -1):
        if T % cand == 0:
            chunk = cand
            break
    if chunk < 8 and T > 32:
        chunk = 32
    n_chunks = -(-T // chunk)
    Tp = n_chunks * chunk
    needs_mask = Tp != T

    emb_tb = embedding[x_idx.T].astype(jnp.bfloat16)
    emb_tb = _pad_to(emb_tb, (Tp, Bp, Hp))
    wih = _pad_to(w_ih_t, (H, Hp)).astype(jnp.bfloat16)
    wih = _pad_to(wih, (Hp, Hp))
    brnn = _pad_to(b_rnn, (1, Hp))
    whh = _pad_to(w_hh_t, (Hp, Hp))
    wfc = _pad_to(w_fc_t, (Hp, Op))
    bfc = _pad_to(b_fc, (1, Op))

    body = _make_body(T, chunk, tb, Hp, needs_mask)

    out_p = pl.pallas_call(
        body,
        grid=(n_btiles, n_chunks),
        in_specs=[
            pl.BlockSpec((chunk, tb, Hp), lambda b, c: (c, b, 0)),
            pl.BlockSpec((Hp, Hp), lambda b, c: (0, 0)),
            pl.BlockSpec((1, Hp), lambda b, c: (0, 0)),
            pl.BlockSpec((Hp, Hp), lambda b, c: (0, 0)),
            pl.BlockSpec((Hp, Op), lambda b, c: (0, 0)),
            pl.BlockSpec((1, Op), lambda b, c: (0, 0)),
        ],
        out_specs=pl.BlockSpec((tb, Op), lambda b, c: (b, 0)),
        out_shape=jax.ShapeDtypeStruct((Bp, Op), jnp.float32),
        scratch_shapes=[
            pltpu.VMEM((tb, Hp), jnp.float32),
            pltpu.VMEM((chunk, tb, Hp), jnp.float32),
        ],
        compiler_params=pltpu.CompilerParams(
            dimension_semantics=("parallel", "arbitrary"),
            vmem_limit_bytes=100 * (1 << 20),
        ),
    )(emb_tb, wih, brnn, whh, wfc, bfc)

    return out_p[:B, :O]
